# baseline (speedup 1.0000x reference)
import functools

import jax
import jax.numpy as jnp
from jax import lax
from jax.experimental import pallas as pl
from jax.experimental.pallas import tpu as pltpu
from jax.experimental.pallas import tpu_sc as plsc

N_GRAPH = 64
N_NODE = 128
N_FEAT = 9
N_ROW = 11
N_PAD = 12
HIDDEN = 768
LANES = 16
NC = 2
NS = 16
NW = NC * NS
GPW = N_GRAPH // NW
C = 4
NCHUNK = N_NODE // C
NT = GPW * NCHUNK
ROWS_PC = C * N_PAD
S = 6
RPS = ROWS_PC // S
IDX_PW = GPW * N_NODE * N_PAD


def _build_kernel():
    mesh = plsc.VectorSubcoreMesh(core_axis_name="c", subcore_axis_name="s")

    @functools.partial(
        pl.kernel,
        mesh=mesh,
        compiler_params=pltpu.CompilerParams(use_tc_tiling_on_sc=False),
        out_type=jax.ShapeDtypeStruct((N_GRAPH, N_NODE + 1, HIDDEN), jnp.float32),
        scratch_types=[
            pltpu.VMEM((IDX_PW,), jnp.int32),
            pltpu.VMEM((2, ROWS_PC, HIDDEN), jnp.float32),
            pltpu.VMEM((2, C, HIDDEN), jnp.float32),
            pltpu.VMEM((1, HIDDEN), jnp.float32),
            pltpu.SemaphoreType.DMA,
            pltpu.SemaphoreType.DMA,
            pltpu.SemaphoreType.DMA,
            pltpu.SemaphoreType.DMA,
            pltpu.SemaphoreType.DMA,
        ],
    )
    def k(idx_hbm, table_hbm, tok_hbm, out_hbm,
          idxv, gbuf, rbuf, tokv, sem_p, sem_g0, sem_g1, sem_o0, sem_o1):
        wid = lax.axis_index("s") * NC + lax.axis_index("c")
        sem_g = (sem_g0, sem_g1)
        sem_o = (sem_o0, sem_o1)

        pltpu.async_copy(idx_hbm.at[pl.ds(wid * IDX_PW, IDX_PW)], idxv, sem_p)
        pltpu.async_copy(tok_hbm, tokv, sem_p)
        pltpu.make_async_copy(idx_hbm.at[pl.ds(0, IDX_PW)], idxv, sem_p).wait()
        pltpu.make_async_copy(tok_hbm, tokv, sem_p).wait()

        for gl in range(GPW):
            g = wid * GPW + gl
            pltpu.async_copy(tokv, out_hbm.at[g, pl.ds(0, 1)], sem_p)

        def fire_gather(t, b):
            for s_ in range(S):
                pltpu.async_copy(
                    table_hbm.at[idxv.at[pl.ds(t * ROWS_PC + s_ * RPS, RPS)]],
                    gbuf.at[b, pl.ds(s_ * RPS, RPS)], sem_g[b])

        def wait_gather(b):
            for s_ in range(S):
                pltpu.make_async_copy(
                    table_hbm.at[idxv.at[pl.ds(s_ * RPS, RPS)]],
                    gbuf.at[b, pl.ds(s_ * RPS, RPS)], sem_g[b]).wait()

        def do_chunk(t, b):
            wait_gather(b)

            @pl.when(t + 1 < NT)
            def _():
                fire_gather(t + 1, 1 - b)

            @pl.when(t >= 2)
            def _():
                pltpu.make_async_copy(
                    rbuf.at[b], out_hbm.at[0, pl.ds(0, C)], sem_o[b]).wait()

            @pl.loop(0, C)
            def _node(i):
                @pl.loop(0, HIDDEN // LANES)
                def _col(j):
                    col = j * LANES
                    acc = gbuf[b, i * N_PAD, pl.ds(col, LANES)]
                    for f in range(1, N_ROW):
                        acc = acc + gbuf[b, i * N_PAD + f, pl.ds(col, LANES)]
                    rbuf[b, i, pl.ds(col, LANES)] = acc

            g = wid * GPW + lax.div(t, NCHUNK)
            node0 = lax.rem(t, NCHUNK) * C
            pltpu.async_copy(rbuf.at[b],
                             out_hbm.at[g, pl.ds(1 + node0, C)], sem_o[b])

        fire_gather(0, 0)

        @pl.loop(0, NT, step=2)
        def _pair(t0):
            do_chunk(t0, 0)
            do_chunk(t0 + 1, 1)

        pltpu.make_async_copy(rbuf.at[0], out_hbm.at[0, pl.ds(0, C)], sem_o[0]).wait()
        pltpu.make_async_copy(rbuf.at[1], out_hbm.at[0, pl.ds(0, C)], sem_o[1]).wait()
        for _ in range(GPW):
            pltpu.make_async_copy(tokv, out_hbm.at[0, pl.ds(0, 1)], sem_p).wait()

    return k


_KERNEL = _build_kernel()


def kernel(x, in_degree, out_degree, atom_table, in_deg_table, out_deg_table,
           graph_token):
    n_atom = atom_table.shape[0]
    n_in = in_deg_table.shape[0]
    x = x.astype(jnp.int32)
    ind = in_degree.astype(jnp.int32) + n_atom
    outd = out_degree.astype(jnp.int32) + n_atom + n_in
    pad = jnp.zeros(x.shape[:2] + (1,), jnp.int32)
    idx = jnp.concatenate([x, ind[..., None], outd[..., None], pad], axis=-1)
    table = jnp.concatenate([atom_table, in_deg_table, out_deg_table], axis=0)
    return _KERNEL(idx.reshape(-1), table, graph_token)


# speedup vs baseline: 1.9644x; 1.9644x over previous
import functools

import jax
import jax.numpy as jnp
from jax import lax
from jax.experimental import pallas as pl
from jax.experimental.pallas import tpu as pltpu
from jax.experimental.pallas import tpu_sc as plsc

N_GRAPH = 64
N_NODE = 128
N_FEAT = 9
N_ROW = 11
HIDDEN = 768
HHID = HIDDEN // 2
BLANES = 32
NC = 2
NS = 16
GPT = N_GRAPH // NS
C = 8
NCHUNK = N_NODE // C
NT = GPT * NCHUNK
ROWS_PC = C * N_ROW
IDX_PT = GPT * N_NODE * N_ROW
N_TAB = 5633
ROWS_PER_TILE = 352


def _build_kernel():
    mesh = plsc.VectorSubcoreMesh(core_axis_name="c", subcore_axis_name="s")

    @functools.partial(
        pl.kernel,
        mesh=mesh,
        compiler_params=pltpu.CompilerParams(use_tc_tiling_on_sc=False),
        out_type=jax.ShapeDtypeStruct((N_GRAPH, N_NODE + 1, HIDDEN),
                                      jnp.bfloat16),
        scratch_types=[
            pltpu.VMEM((IDX_PT,), jnp.int32),
            pltpu.VMEM((2, ROWS_PC, HHID), jnp.bfloat16),
            pltpu.VMEM((2, C, HHID), jnp.bfloat16),
            pltpu.VMEM((1, HHID), jnp.bfloat16),
            pltpu.VMEM_SHARED((N_TAB, HHID), jnp.bfloat16),
            pltpu.SemaphoreType.DMA,
            pltpu.SemaphoreType.DMA,
            pltpu.SemaphoreType.DMA,
            pltpu.SemaphoreType.DMA,
            pltpu.SemaphoreType.DMA,
        ],
    )
    def k(idx_hbm, tlo_hbm, thi_hbm, tok_hbm, out_hbm,
          idxv, gbuf, rbuf, tokv, spt,
          sem_p, sem_g0, sem_g1, sem_o0, sem_o1):
        cid = lax.axis_index("c")
        sid = lax.axis_index("s")
        sem_g = (sem_g0, sem_g1)
        sem_o = (sem_o0, sem_o1)
        col0 = cid * HHID

        r0 = sid * ROWS_PER_TILE

        @pl.when(cid == 0)
        def _():
            pltpu.async_copy(tlo_hbm.at[pl.ds(r0, ROWS_PER_TILE)],
                             spt.at[pl.ds(r0, ROWS_PER_TILE)], sem_p)

        @pl.when(cid == 1)
        def _():
            pltpu.async_copy(thi_hbm.at[pl.ds(r0, ROWS_PER_TILE)],
                             spt.at[pl.ds(r0, ROWS_PER_TILE)], sem_p)

        @pl.when(jnp.logical_and(sid == 0, cid == 0))
        def _():
            pltpu.async_copy(tlo_hbm.at[pl.ds(N_TAB - 1, 1)],
                             spt.at[pl.ds(N_TAB - 1, 1)], sem_p)

        @pl.when(jnp.logical_and(sid == 0, cid == 1))
        def _():
            pltpu.async_copy(thi_hbm.at[pl.ds(N_TAB - 1, 1)],
                             spt.at[pl.ds(N_TAB - 1, 1)], sem_p)

        pltpu.async_copy(idx_hbm.at[pl.ds(sid * IDX_PT, IDX_PT)], idxv, sem_p)
        pltpu.async_copy(tok_hbm.at[pl.ds(0, 1), pl.ds(col0, HHID)], tokv,
                         sem_p)

        pltpu.make_async_copy(idx_hbm.at[pl.ds(0, IDX_PT)], idxv, sem_p).wait()
        pltpu.make_async_copy(tok_hbm.at[pl.ds(0, 1), pl.ds(0, HHID)], tokv,
                              sem_p).wait()
        pltpu.make_async_copy(
            tlo_hbm.at[pl.ds(0, ROWS_PER_TILE)],
            spt.at[pl.ds(0, ROWS_PER_TILE)], sem_p).wait()

        @pl.when(sid == 0)
        def _():
            pltpu.make_async_copy(tlo_hbm.at[pl.ds(0, 1)],
                                  spt.at[pl.ds(0, 1)], sem_p).wait()

        plsc.subcore_barrier()

        for gl in range(GPT):
            g = sid * GPT + gl
            pltpu.async_copy(tokv,
                             out_hbm.at[g, pl.ds(0, 1), pl.ds(col0, HHID)],
                             sem_p)

        def fire_gather(t, b):
            pltpu.async_copy(
                spt.at[idxv.at[pl.ds(t * ROWS_PC, ROWS_PC)]],
                gbuf.at[b], sem_g[b])

        def do_chunk(t, b):
            pltpu.make_async_copy(
                spt.at[idxv.at[pl.ds(0, ROWS_PC)]],
                gbuf.at[b], sem_g[b]).wait()

            @pl.when(t + 1 < NT)
            def _():
                fire_gather(t + 1, 1 - b)

            @pl.when(t >= 2)
            def _():
                pltpu.make_async_copy(
                    rbuf.at[b],
                    out_hbm.at[0, pl.ds(0, C), pl.ds(0, HHID)],
                    sem_o[b]).wait()

            @pl.loop(0, C)
            def _node(i):
                @pl.loop(0, HHID // BLANES)
                def _col(j):
                    col = j * BLANES
                    acc = gbuf[b, i * N_ROW, pl.ds(col, BLANES)]
                    for f in range(1, N_ROW):
                        acc = acc + gbuf[b, i * N_ROW + f, pl.ds(col, BLANES)]
                    rbuf[b, i, pl.ds(col, BLANES)] = acc

            g = sid * GPT + lax.div(t, NCHUNK)
            node0 = lax.rem(t, NCHUNK) * C
            pltpu.async_copy(
                rbuf.at[b],
                out_hbm.at[g, pl.ds(1 + node0, C), pl.ds(col0, HHID)],
                sem_o[b])

        fire_gather(0, 0)

        @pl.loop(0, NT, step=2)
        def _pair(t0):
            do_chunk(t0, 0)
            do_chunk(t0 + 1, 1)

        pltpu.make_async_copy(
            rbuf.at[0], out_hbm.at[0, pl.ds(0, C), pl.ds(0, HHID)],
            sem_o[0]).wait()
        pltpu.make_async_copy(
            rbuf.at[1], out_hbm.at[0, pl.ds(0, C), pl.ds(0, HHID)],
            sem_o[1]).wait()
        for _ in range(GPT):
            pltpu.make_async_copy(
                tokv, out_hbm.at[0, pl.ds(0, 1), pl.ds(0, HHID)],
                sem_p).wait()

    return k


_KERNEL = _build_kernel()


def kernel(x, in_degree, out_degree, atom_table, in_deg_table, out_deg_table,
           graph_token):
    n_atom = atom_table.shape[0]
    n_in = in_deg_table.shape[0]
    x = x.astype(jnp.int32)
    ind = in_degree.astype(jnp.int32) + n_atom
    outd = out_degree.astype(jnp.int32) + n_atom + n_in
    idx = jnp.concatenate([x, ind[..., None], outd[..., None]], axis=-1)
    table = jnp.concatenate([atom_table, in_deg_table, out_deg_table],
                            axis=0).astype(jnp.bfloat16)
    tlo = table[:, :HHID].copy()
    thi = table[:, HHID:].copy()
    out = _KERNEL(idx.reshape(-1), tlo, thi,
                  graph_token.astype(jnp.bfloat16))
    return out.astype(jnp.float32)


# speedup vs baseline: 2.4334x; 1.2388x over previous
import functools

import jax
import jax.numpy as jnp
from jax import lax
from jax.experimental import pallas as pl
from jax.experimental.pallas import tpu as pltpu
from jax.experimental.pallas import tpu_sc as plsc

N_GRAPH = 64
N_NODE = 128
N_FEAT = 9
N_ROW = 11
HIDDEN = 768
HHID = HIDDEN // 2
BLANES = 32
NC = 2
NS = 16
GPT = N_GRAPH // NS
C = 8
NCHUNK = N_NODE // C
NT = GPT * NCHUNK
ROWS_PC = C * N_ROW
IDX_PT = GPT * N_NODE * N_ROW
NBUF = 2

N_ATOM = 4609
N_DEG = 512
IN_OFF = 4616
OUT_OFF = IN_OFF + N_DEG + 8
N_TAB = OUT_OFF + N_DEG
ATOM_RPT = 288
DEG_RPT = N_DEG // NS


def _build_kernel():
    mesh = plsc.VectorSubcoreMesh(core_axis_name="c", subcore_axis_name="s")

    @functools.partial(
        pl.kernel,
        mesh=mesh,
        compiler_params=pltpu.CompilerParams(use_tc_tiling_on_sc=False),
        out_type=jax.ShapeDtypeStruct((N_GRAPH, N_NODE + 1, HIDDEN),
                                      jnp.bfloat16),
        scratch_types=[
            pltpu.VMEM((IDX_PT,), jnp.int32),
            pltpu.VMEM((NBUF, ROWS_PC, HHID), jnp.bfloat16),
            pltpu.VMEM((NBUF, C, HHID), jnp.bfloat16),
            pltpu.VMEM((1, HHID), jnp.bfloat16),
            pltpu.VMEM_SHARED((N_TAB, HHID), jnp.bfloat16),
            pltpu.SemaphoreType.DMA,
            pltpu.SemaphoreType.DMA,
            pltpu.SemaphoreType.DMA,
            pltpu.SemaphoreType.DMA,
            pltpu.SemaphoreType.DMA,
        ],
    )
    def k(idx_hbm, atom_hbm, ind_hbm, outd_hbm, tok_hbm, out_hbm,
          idxv, gbuf, rbuf, tokv, spt,
          sem_p, sem_g0, sem_g1, sem_o0, sem_o1):
        cid = lax.axis_index("c")
        sid = lax.axis_index("s")
        sem_g = (sem_g0, sem_g1)
        sem_o = (sem_o0, sem_o1)
        col0 = cid * HHID

        ar0 = sid * ATOM_RPT
        dr0 = sid * DEG_RPT
        pltpu.async_copy(atom_hbm.at[pl.ds(ar0, ATOM_RPT), pl.ds(col0, HHID)],
                         spt.at[pl.ds(ar0, ATOM_RPT)], sem_p)
        pltpu.async_copy(ind_hbm.at[pl.ds(dr0, DEG_RPT), pl.ds(col0, HHID)],
                         spt.at[pl.ds(IN_OFF + dr0, DEG_RPT)], sem_p)
        pltpu.async_copy(outd_hbm.at[pl.ds(dr0, DEG_RPT), pl.ds(col0, HHID)],
                         spt.at[pl.ds(OUT_OFF + dr0, DEG_RPT)], sem_p)

        @pl.when(sid == 0)
        def _():
            pltpu.async_copy(
                atom_hbm.at[pl.ds(N_ATOM - 1, 1), pl.ds(col0, HHID)],
                spt.at[pl.ds(N_ATOM - 1, 1)], sem_p)

        pltpu.async_copy(idx_hbm.at[pl.ds(sid * IDX_PT, IDX_PT)], idxv, sem_p)
        pltpu.async_copy(tok_hbm.at[pl.ds(0, 1), pl.ds(col0, HHID)], tokv,
                         sem_p)

        pltpu.make_async_copy(
            atom_hbm.at[pl.ds(0, ATOM_RPT), pl.ds(0, HHID)],
            spt.at[pl.ds(0, ATOM_RPT)], sem_p).wait()
        pltpu.make_async_copy(
            ind_hbm.at[pl.ds(0, DEG_RPT), pl.ds(0, HHID)],
            spt.at[pl.ds(IN_OFF, DEG_RPT)], sem_p).wait()
        pltpu.make_async_copy(
            outd_hbm.at[pl.ds(0, DEG_RPT), pl.ds(0, HHID)],
            spt.at[pl.ds(OUT_OFF, DEG_RPT)], sem_p).wait()
        pltpu.make_async_copy(idx_hbm.at[pl.ds(0, IDX_PT)], idxv, sem_p).wait()
        pltpu.make_async_copy(tok_hbm.at[pl.ds(0, 1), pl.ds(0, HHID)], tokv,
                              sem_p).wait()

        @pl.when(sid == 0)
        def _():
            pltpu.make_async_copy(
                atom_hbm.at[pl.ds(0, 1), pl.ds(0, HHID)],
                spt.at[pl.ds(0, 1)], sem_p).wait()

        plsc.subcore_barrier()

        for gl in range(GPT):
            g = sid * GPT + gl
            pltpu.async_copy(tokv,
                             out_hbm.at[g, pl.ds(0, 1), pl.ds(col0, HHID)],
                             sem_p)

        def fire_gather(t, b):
            pltpu.async_copy(
                spt.at[idxv.at[pl.ds(t * ROWS_PC, ROWS_PC)]],
                gbuf.at[b], sem_g[b])

        def do_chunk(t, b):
            pltpu.make_async_copy(
                spt.at[idxv.at[pl.ds(0, ROWS_PC)]],
                gbuf.at[b], sem_g[b]).wait()

            @pl.when(t + NBUF - 1 < NT)
            def _():
                fire_gather(t + NBUF - 1, (b + NBUF - 1) % NBUF)

            @pl.when(t >= NBUF)
            def _():
                pltpu.make_async_copy(
                    rbuf.at[b],
                    out_hbm.at[0, pl.ds(0, C), pl.ds(0, HHID)],
                    sem_o[b]).wait()

            @pl.loop(0, C)
            def _node(i):
                @pl.loop(0, HHID // BLANES)
                def _col(j):
                    col = j * BLANES
                    acc = gbuf[b, i * N_ROW, pl.ds(col, BLANES)]
                    for f in range(1, N_ROW):
                        acc = acc + gbuf[b, i * N_ROW + f, pl.ds(col, BLANES)]
                    rbuf[b, i, pl.ds(col, BLANES)] = acc

            g = sid * GPT + lax.div(t, NCHUNK)
            node0 = lax.rem(t, NCHUNK) * C
            pltpu.async_copy(
                rbuf.at[b],
                out_hbm.at[g, pl.ds(1 + node0, C), pl.ds(col0, HHID)],
                sem_o[b])

        for b in range(NBUF - 1):
            fire_gather(b, b)

        @pl.loop(0, NT, step=NBUF)
        def _quad(t0):
            for b in range(NBUF):
                do_chunk(t0 + b, b)

        for b in range(NBUF):
            pltpu.make_async_copy(
                rbuf.at[b], out_hbm.at[0, pl.ds(0, C), pl.ds(0, HHID)],
                sem_o[b]).wait()
        for _ in range(GPT):
            pltpu.make_async_copy(
                tokv, out_hbm.at[0, pl.ds(0, 1), pl.ds(0, HHID)],
                sem_p).wait()

    return k


_KERNEL = _build_kernel()


def kernel(x, in_degree, out_degree, atom_table, in_deg_table, out_deg_table,
           graph_token):
    idx = jnp.concatenate(
        [x.astype(jnp.int32),
         in_degree.astype(jnp.int32)[..., None] + IN_OFF,
         out_degree.astype(jnp.int32)[..., None] + OUT_OFF], axis=-1)
    out = _KERNEL(idx.reshape(-1),
                  atom_table.astype(jnp.bfloat16),
                  in_deg_table.astype(jnp.bfloat16),
                  out_deg_table.astype(jnp.bfloat16),
                  graph_token.astype(jnp.bfloat16))
    return out.astype(jnp.float32)
